# baseline (speedup 1.0000x reference)
import functools

import jax
import jax.numpy as jnp
from jax import lax
from jax.experimental import pallas as pl
from jax.experimental.pallas import tpu as pltpu

N_MESH = 10000
N_GRID = 50000
E = 160000
D = 256
H = 256
EO = 256

_EPS = 1e-5


def _ln_rows(h, g, b):
    mu = jnp.mean(h, axis=-1, keepdims=True)
    var = jnp.mean((h - mu) * (h - mu), axis=-1, keepdims=True)
    return (h - mu) * lax.rsqrt(var + _EPS) * g + b


def _matmul_body(x_ref, w_ref, o_ref):
    o_ref[...] = jnp.dot(x_ref[...], w_ref[...],
                         preferred_element_type=jnp.float32)


def _matmul(x, w, block):
    m, k = x.shape
    _, n = w.shape
    return pl.pallas_call(
        _matmul_body,
        grid=(m // block,),
        in_specs=[
            pl.BlockSpec((block, k), lambda i: (i, 0)),
            pl.BlockSpec((k, n), lambda i: (0, 0)),
        ],
        out_specs=pl.BlockSpec((block, n), lambda i: (i, 0)),
        out_shape=jax.ShapeDtypeStruct((m, n), jnp.float32),
    )(x, w)


def _edge_mlp_body(em_ref, gs_ref, gd_ref, we_ref, be_ref, weo_ref, beo_ref,
                   ge_ref, ble_ref, o_ref):
    h = jnp.dot(em_ref[...], we_ref[...], preferred_element_type=jnp.float32)
    h = h + gs_ref[...] + gd_ref[...] + be_ref[...]
    h = h * jax.nn.sigmoid(h)
    h = jnp.dot(h, weo_ref[...], preferred_element_type=jnp.float32)
    h = h + beo_ref[...]
    o_ref[...] = _ln_rows(h, ge_ref[...], ble_ref[...])


def _edge_mlp(em, gsrc, gdst, we, be, weo, beo, ge, ble, block=640):
    e = em.shape[0]
    row = pl.BlockSpec((block, D), lambda i: (i, 0))
    mat = pl.BlockSpec((D, H), lambda i: (0, 0))
    vec = pl.BlockSpec((1, H), lambda i: (0, 0))
    return pl.pallas_call(
        _edge_mlp_body,
        grid=(e // block,),
        in_specs=[row, row, row, mat, vec, mat, vec, vec, vec],
        out_specs=pl.BlockSpec((block, EO), lambda i: (i, 0)),
        out_shape=jax.ShapeDtypeStruct((e, EO), jnp.float32),
    )(em, gsrc, gdst, we, be.reshape(1, H), weo, beo.reshape(1, EO),
      ge.reshape(1, EO), ble.reshape(1, EO))


def _node_mlp_body(agg_ref, grid_ref, w0a_ref, w0b_ref, b0_ref, w1_ref,
                   b1_ref, gn_ref, bln_ref, o_ref):
    n = jnp.dot(agg_ref[...], w0a_ref[...], preferred_element_type=jnp.float32)
    n = n + jnp.dot(grid_ref[...], w0b_ref[...],
                    preferred_element_type=jnp.float32)
    n = n + b0_ref[...]
    n = n * jax.nn.sigmoid(n)
    n = jnp.dot(n, w1_ref[...], preferred_element_type=jnp.float32)
    n = n + b1_ref[...]
    o_ref[...] = _ln_rows(n, gn_ref[...], bln_ref[...]) + grid_ref[...]


def _node_mlp(agg, grid, w0a, w0b, b0, w1, b1, gn, bln, block=1000):
    m = agg.shape[0]
    row = pl.BlockSpec((block, D), lambda i: (i, 0))
    mat = pl.BlockSpec((D, H), lambda i: (0, 0))
    vec = pl.BlockSpec((1, H), lambda i: (0, 0))
    return pl.pallas_call(
        _node_mlp_body,
        grid=(m // block,),
        in_specs=[row, row, mat, mat, vec,
                  pl.BlockSpec((H, D), lambda i: (0, 0)), vec, vec, vec],
        out_specs=pl.BlockSpec((block, D), lambda i: (i, 0)),
        out_shape=jax.ShapeDtypeStruct((m, D), jnp.float32),
    )(agg, grid, w0a, w0b, b0.reshape(1, H), w1, b1.reshape(1, D),
      gn.reshape(1, D), bln.reshape(1, D))


def kernel(edge_index, m2g_edge_embedded, m2m_node_processed,
           grid_input_encoded, We, Wsrc, Wdst, be, Weo, beo, ge, ble,
           Wn0, bn0, Wn1, bn1, gn, bln):
    src = edge_index[0]
    dst = edge_index[1]

    h_src = _matmul(m2m_node_processed, Wsrc, 1000)
    h_dst = _matmul(grid_input_encoded, Wdst, 1000)

    gsrc = jnp.take(h_src, src, axis=0)
    gdst = jnp.take(h_dst, dst, axis=0)

    efeat = _edge_mlp(m2g_edge_embedded, gsrc, gdst, We, be, Weo, beo, ge, ble)

    agg = jax.ops.segment_sum(efeat, dst, num_segments=N_GRID)

    w0a = Wn0[:D]
    w0b = Wn0[D:]
    return _node_mlp(agg, grid_input_encoded, w0a, w0b, bn0, Wn1, bn1, gn, bln)


# speedup vs baseline: 1.6257x; 1.6257x over previous
import functools

import jax
import jax.numpy as jnp
from jax import lax
from jax.experimental import pallas as pl
from jax.experimental.pallas import tpu as pltpu
from jax.experimental.pallas import tpu_sc as plsc

N_MESH = 10000
N_GRID = 50000
E = 160000
D = 256
H = 256
EO = 256

_EPS = 1e-5


def _ln_rows(h, g, b):
    mu = jnp.mean(h, axis=-1, keepdims=True)
    var = jnp.mean((h - mu) * (h - mu), axis=-1, keepdims=True)
    return (h - mu) * lax.rsqrt(var + _EPS) * g + b


def _matmul_body(x_ref, w_ref, o_ref):
    o_ref[...] = jnp.dot(x_ref[...], w_ref[...],
                         preferred_element_type=jnp.float32)


def _matmul(x, w, block):
    m, k = x.shape
    _, n = w.shape
    return pl.pallas_call(
        _matmul_body,
        grid=(m // block,),
        in_specs=[
            pl.BlockSpec((block, k), lambda i: (i, 0)),
            pl.BlockSpec((k, n), lambda i: (0, 0)),
        ],
        out_specs=pl.BlockSpec((block, n), lambda i: (i, 0)),
        out_shape=jax.ShapeDtypeStruct((m, n), jnp.float32),
    )(x, w)


_NC = 2
_NS = 16
_NW = _NC * _NS
_EPW = E // _NW
_BR = 128
_NFB = _EPW // _BR
_REM = _EPW - _NFB * _BR


def _sc_gather_body(src_hbm, dst_hbm, hsrc_hbm, hdst_hbm, gs_hbm, gd_hbm,
                    src_v, dst_v, rows_s, rows_d, sem_s, sem_d):
    wid = lax.axis_index("s") * _NC + lax.axis_index("c")
    base = pl.multiple_of(wid * _EPW, 8)
    pltpu.sync_copy(src_hbm.at[pl.ds(base, _EPW)], src_v)
    pltpu.sync_copy(dst_hbm.at[pl.ds(base, _EPW)], dst_v)

    def body(j, carry):
        off = pl.multiple_of(j * _BR, 8)
        cs = pltpu.async_copy(hsrc_hbm.at[src_v.at[pl.ds(off, _BR)]],
                              rows_s, sem_s)
        cd = pltpu.async_copy(hdst_hbm.at[dst_v.at[pl.ds(off, _BR)]],
                              rows_d, sem_d)
        cs.wait()
        pltpu.sync_copy(rows_s, gs_hbm.at[pl.ds(base + off, _BR)])
        cd.wait()
        pltpu.sync_copy(rows_d, gd_hbm.at[pl.ds(base + off, _BR)])
        return carry

    lax.fori_loop(0, _NFB, body, 0)
    off = _NFB * _BR
    cs = pltpu.async_copy(hsrc_hbm.at[src_v.at[pl.ds(off, _REM)]],
                          rows_s.at[pl.ds(0, _REM)], sem_s)
    cd = pltpu.async_copy(hdst_hbm.at[dst_v.at[pl.ds(off, _REM)]],
                          rows_d.at[pl.ds(0, _REM)], sem_d)
    cs.wait()
    pltpu.sync_copy(rows_s.at[pl.ds(0, _REM)], gs_hbm.at[pl.ds(base + off, _REM)])
    cd.wait()
    pltpu.sync_copy(rows_d.at[pl.ds(0, _REM)], gd_hbm.at[pl.ds(base + off, _REM)])


def _sc_gather(src, dst, h_src, h_dst):
    mesh = plsc.VectorSubcoreMesh(core_axis_name="c", subcore_axis_name="s")
    f = pl.kernel(
        _sc_gather_body,
        out_type=(jax.ShapeDtypeStruct((E, H), jnp.float32),
                  jax.ShapeDtypeStruct((E, H), jnp.float32)),
        mesh=mesh,
        scratch_types=[
            pltpu.VMEM((_EPW,), jnp.int32),
            pltpu.VMEM((_EPW,), jnp.int32),
            pltpu.VMEM((_BR, H), jnp.float32),
            pltpu.VMEM((_BR, H), jnp.float32),
            pltpu.SemaphoreType.DMA,
            pltpu.SemaphoreType.DMA,
        ],
    )
    return f(src, dst, h_src, h_dst)


def _edge_mlp_body(em_ref, gs_ref, gd_ref, we_ref, be_ref, weo_ref, beo_ref,
                   ge_ref, ble_ref, o_ref):
    h = jnp.dot(em_ref[...], we_ref[...], preferred_element_type=jnp.float32)
    h = h + gs_ref[...] + gd_ref[...] + be_ref[...]
    h = h * jax.nn.sigmoid(h)
    h = jnp.dot(h, weo_ref[...], preferred_element_type=jnp.float32)
    h = h + beo_ref[...]
    o_ref[...] = _ln_rows(h, ge_ref[...], ble_ref[...])


def _edge_mlp(em, gsrc, gdst, we, be, weo, beo, ge, ble, block=640):
    e = em.shape[0]
    row = pl.BlockSpec((block, D), lambda i: (i, 0))
    mat = pl.BlockSpec((D, H), lambda i: (0, 0))
    vec = pl.BlockSpec((1, H), lambda i: (0, 0))
    return pl.pallas_call(
        _edge_mlp_body,
        grid=(e // block,),
        in_specs=[row, row, row, mat, vec, mat, vec, vec, vec],
        out_specs=pl.BlockSpec((block, EO), lambda i: (i, 0)),
        out_shape=jax.ShapeDtypeStruct((e, EO), jnp.float32),
    )(em, gsrc, gdst, we, be.reshape(1, H), weo, beo.reshape(1, EO),
      ge.reshape(1, EO), ble.reshape(1, EO))


def _node_mlp_body(agg_ref, grid_ref, w0a_ref, w0b_ref, b0_ref, w1_ref,
                   b1_ref, gn_ref, bln_ref, o_ref):
    n = jnp.dot(agg_ref[...], w0a_ref[...], preferred_element_type=jnp.float32)
    n = n + jnp.dot(grid_ref[...], w0b_ref[...],
                    preferred_element_type=jnp.float32)
    n = n + b0_ref[...]
    n = n * jax.nn.sigmoid(n)
    n = jnp.dot(n, w1_ref[...], preferred_element_type=jnp.float32)
    n = n + b1_ref[...]
    o_ref[...] = _ln_rows(n, gn_ref[...], bln_ref[...]) + grid_ref[...]


def _node_mlp(agg, grid, w0a, w0b, b0, w1, b1, gn, bln, block=1000):
    m = agg.shape[0]
    row = pl.BlockSpec((block, D), lambda i: (i, 0))
    mat = pl.BlockSpec((D, H), lambda i: (0, 0))
    vec = pl.BlockSpec((1, H), lambda i: (0, 0))
    return pl.pallas_call(
        _node_mlp_body,
        grid=(m // block,),
        in_specs=[row, row, mat, mat, vec,
                  pl.BlockSpec((H, D), lambda i: (0, 0)), vec, vec, vec],
        out_specs=pl.BlockSpec((block, D), lambda i: (i, 0)),
        out_shape=jax.ShapeDtypeStruct((m, D), jnp.float32),
    )(agg, grid, w0a, w0b, b0.reshape(1, H), w1, b1.reshape(1, D),
      gn.reshape(1, D), bln.reshape(1, D))


def kernel(edge_index, m2g_edge_embedded, m2m_node_processed,
           grid_input_encoded, We, Wsrc, Wdst, be, Weo, beo, ge, ble,
           Wn0, bn0, Wn1, bn1, gn, bln):
    src = edge_index[0]
    dst = edge_index[1]

    h_src = _matmul(m2m_node_processed, Wsrc, 1000)
    h_dst = _matmul(grid_input_encoded, Wdst, 1000)

    gsrc, gdst = _sc_gather(src, dst, h_src, h_dst)

    efeat = _edge_mlp(m2g_edge_embedded, gsrc, gdst, We, be, Weo, beo, ge, ble)

    agg = jax.ops.segment_sum(efeat, dst, num_segments=N_GRID)

    w0a = Wn0[:D]
    w0b = Wn0[D:]
    return _node_mlp(agg, grid_input_encoded, w0a, w0b, bn0, Wn1, bn1, gn, bln)
